# baseline (speedup 1.0000x reference)
import functools

import jax
import jax.numpy as jnp
from jax import lax
from jax.experimental import pallas as pl
from jax.experimental.pallas import tpu as pltpu
from jax.experimental.pallas import tpu_sc as plsc

N = 10000
E = 320000
D = 128
B = 4096

NC = 2
NS = 16
NW = NC * NS
L = 16

C = 96
NCH = 108
EPT = NCH * C
EPAD = NW * EPT

NP = 10240
ROWS_PER_TILE = NP // NS

BR = 1024
PAIRS_PER_TILE = B // NW


def _sc_mesh():
    return plsc.VectorSubcoreMesh(
        core_axis_name="c", subcore_axis_name="s", num_cores=NC,
        num_subcores=NS)


_SC_PARAMS = pltpu.CompilerParams(needs_layout_passes=False)


NR = NP // 128


def _deg_body(src_hbm, dst_hbm, w_hbm, out_hbm,
              sslab, dslab, wslab, deg_s, deg_d, ids0, ids1, shacc):
    c = lax.axis_index("c")
    s = lax.axis_index("s")
    wid = c * NS + s

    pltpu.sync_copy(src_hbm.at[wid], sslab)
    pltpu.sync_copy(dst_hbm.at[wid], dslab)
    pltpu.sync_copy(w_hbm.at[wid], wslab)

    def zero_body(r, _):
        z = jnp.zeros((L,), jnp.float32)
        for k in range(128 // L):
            deg_s[r, pl.ds(k * L, L)] = z
            deg_d[r, pl.ds(k * L, L)] = z
        return _

    lax.fori_loop(0, NR, zero_body, None)

    def ids_body(j, _):
        v = jnp.arange(L, dtype=jnp.int32) + j * L
        ids0[pl.ds(j * L, L)] = v
        ids1[pl.ds(j * L, L)] = v + NR
        return _

    lax.fori_loop(0, NR // L, ids_body, None)

    @pl.when(s == 0)
    def _zero_shared():
        pltpu.sync_copy(deg_s, shacc.at[pl.ds(0, NR)])
        pltpu.sync_copy(deg_s, shacc.at[pl.ds(NR, NR)])

    def chunk_body(g, _):
        for j in range(C // L):
            sl = pl.ds(j * L, L)
            wv = wslab[g, sl]
            si = sslab[g, sl]
            di = dslab[g, sl]
            plsc.addupdate_scatter(
                deg_s,
                [lax.shift_right_logical(si, 7), jnp.bitwise_and(si, 127)],
                wv)
            plsc.addupdate_scatter(
                deg_d,
                [lax.shift_right_logical(di, 7), jnp.bitwise_and(di, 127)],
                wv)
        return _

    plsc.subcore_barrier()
    lax.fori_loop(0, NCH, chunk_body, None)
    pltpu.sync_copy(deg_s, shacc.at[ids0], add=True)
    pltpu.sync_copy(deg_d, shacc.at[ids1], add=True)
    plsc.subcore_barrier()

    @pl.when(s == 0)
    def _writeout():
        pltpu.sync_copy(shacc, out_hbm.at[c])


def _sc_degrees(srcp, dstp, wp):
    return pl.kernel(
        _deg_body,
        out_type=jax.ShapeDtypeStruct((NC, 2 * NR, 128), jnp.float32),
        mesh=_sc_mesh(),
        compiler_params=_SC_PARAMS,
        scratch_types=[
            pltpu.VMEM((NCH, C), jnp.int32),
            pltpu.VMEM((NCH, C), jnp.int32),
            pltpu.VMEM((NCH, C), jnp.float32),
            pltpu.VMEM((NR, 128), jnp.float32),
            pltpu.VMEM((NR, 128), jnp.float32),
            pltpu.VMEM((NR,), jnp.int32),
            pltpu.VMEM((NR,), jnp.int32),
            pltpu.VMEM_SHARED((2 * NR, 128), jnp.float32),
        ],
    )(srcp, dstp, wp)


def _mm_body(x_ref, w_ref, out_ref):
    out_ref[...] = jnp.dot(x_ref[...], w_ref[...],
                           preferred_element_type=jnp.float32)


def _tc_mm(xp, W):
    grid = NP // BR
    return pl.pallas_call(
        _mm_body,
        grid=(grid,),
        in_specs=[
            pl.BlockSpec((BR, D), lambda i: (i, 0)),
            pl.BlockSpec((D, D), lambda i: (0, 0)),
        ],
        out_specs=pl.BlockSpec((BR, D), lambda i: (i, 0)),
        out_shape=jax.ShapeDtypeStruct((NP, D), jnp.float32),
    )(xp, W)


def _rsqrt_scale_body(degp_ref, xw_ref, g_ref, rr_ref):
    d = jnp.sum(degp_ref[...], axis=0)
    rr = lax.rsqrt(jnp.maximum(d, 1e-6))
    rr_ref[...] = rr
    g_ref[...] = xw_ref[...] * jnp.reshape(rr[0], (BR, 1))


def _tc_rsqrt_scale(degp, xw1):
    grid = NP // BR
    return pl.pallas_call(
        _rsqrt_scale_body,
        grid=(grid,),
        in_specs=[
            pl.BlockSpec((NC, 2, BR), lambda i: (0, 0, i)),
            pl.BlockSpec((BR, D), lambda i: (i, 0)),
        ],
        out_specs=[
            pl.BlockSpec((BR, D), lambda i: (i, 0)),
            pl.BlockSpec((2, BR), lambda i: (0, i)),
        ],
        out_shape=[
            jax.ShapeDtypeStruct((NP, D), jnp.float32),
            jax.ShapeDtypeStruct((2, NP), jnp.float32),
        ],
    )(degp, xw1)


def _edge_body(g_hbm, src_hbm, dst_hbm, w_hbm, out_hbm,
               sidx0, didx0, wch0, sidx1, didx1, wch1,
               sidx2, didx2, wch2, sidx3, didx3, wch3,
               rows0, rows1, rows2, acc,
               sI0, sI1, sI2, sI3, sG0, sG1, sG2, sA0, sA1, sA2):
    c = lax.axis_index("c")
    s = lax.axis_index("s")
    wid = c * NS + s

    small = (
        (sidx0, didx0, wch0, sI0),
        (sidx1, didx1, wch1, sI1),
        (sidx2, didx2, wch2, sI2),
        (sidx3, didx3, wch3, sI3),
    )
    rowsets = ((rows0, sG0, sA0), (rows1, sG1, sA1), (rows2, sG2, sA2))

    def zbuf_body(r, _):
        z = jnp.zeros((L,), jnp.float32)
        for k in range(D // L):
            rows0[r, pl.ds(k * L, L)] = z
        return _

    lax.fori_loop(0, C, zbuf_body, None)
    base = s * ROWS_PER_TILE
    nfull = ROWS_PER_TILE // C
    for r in range(nfull):
        pltpu.sync_copy(rows0, acc.at[pl.ds(base + r * C, C)])
    rem = ROWS_PER_TILE - nfull * C
    if rem:
        pltpu.sync_copy(rows0.at[pl.ds(0, rem)],
                        acc.at[pl.ds(base + nfull * C, rem)])
    plsc.subcore_barrier()

    def startI(gc, sm):
        sidx, didx, wch, sI = sm
        pltpu.async_copy(src_hbm.at[wid, gc], sidx, sI)
        pltpu.async_copy(dst_hbm.at[wid, gc], didx, sI)
        pltpu.async_copy(w_hbm.at[wid, gc], wch, sI)

    def startG(gc, sm, rs):
        sidx, didx, wch, sI = sm
        rows, sG, sA = rs
        pltpu.make_async_copy(src_hbm.at[wid, gc], sidx, sI).wait()
        pltpu.make_async_copy(dst_hbm.at[wid, gc], didx, sI).wait()
        pltpu.make_async_copy(w_hbm.at[wid, gc], wch, sI).wait()
        pltpu.async_copy(g_hbm.at[sidx], rows, sG)

    def proc(gc, sm, rs):
        sidx, didx, wch, sI = sm
        rows, sG, sA = rs
        pltpu.make_async_copy(g_hbm.at[sidx], rows, sG).wait()

        def scale_body(e16, _):
            wv = wch[pl.ds(e16 * L, L)]
            for j in range(L):
                e = e16 * L + j
                we = wv[j]
                for k in range(D // L):
                    sl = pl.ds(k * L, L)
                    rows[e, sl] = rows[e, sl] * we
            return _

        lax.fori_loop(0, C // L, scale_body, None)
        pltpu.async_copy(rows, acc.at[didx], sA, add=True)

    def drainA(sm, rs):
        sidx, didx, wch, sI = sm
        rows, sG, sA = rs
        pltpu.make_async_copy(rows, acc.at[didx], sA).wait()

    startI(0, small[0])
    startI(1, small[1])
    startG(0, small[0], rowsets[0])

    def slot12(i, _):
        for p in range(12):
            t = 12 * i + p

            @pl.when(t + 2 < NCH)
            def _i():
                startI(t + 2, small[(p + 2) % 4])

            @pl.when(t + 1 < NCH)
            def _g():
                startG(t + 1, small[(p + 1) % 4], rowsets[(p + 1) % 3])

            proc(t, small[p % 4], rowsets[p % 3])

            @pl.when(t >= 1)
            def _d():
                drainA(small[(p + 3) % 4], rowsets[(p + 2) % 3])

        return _

    lax.fori_loop(0, NCH // 12, slot12, None)
    drainA(small[(NCH - 1) % 4], rowsets[(NCH - 1) % 3])
    plsc.subcore_barrier()
    pltpu.sync_copy(acc.at[pl.ds(base, ROWS_PER_TILE)],
                    out_hbm.at[c, pl.ds(base, ROWS_PER_TILE)])


def _sc_edge_pass(g, srcp, dstp, wp):
    small_set = [
        pltpu.VMEM((C,), jnp.int32),
        pltpu.VMEM((C,), jnp.int32),
        pltpu.VMEM((C,), jnp.float32),
    ]
    return pl.kernel(
        _edge_body,
        out_type=jax.ShapeDtypeStruct((NC, NP, D), jnp.float32),
        mesh=_sc_mesh(),
        compiler_params=_SC_PARAMS,
        scratch_types=(small_set * 4
                       + [pltpu.VMEM((C, D), jnp.float32)] * 3
                       + [pltpu.VMEM_SHARED((NP, D), jnp.float32)]
                       + [pltpu.SemaphoreType.DMA] * 10),
    )(g, srcp, dstp, wp)


def _mid_body(a_ref, b_ref, rd_ref, rs_ref, b1_ref, w_ref, out_ref):
    h = (a_ref[...] + b_ref[...]) * rd_ref[...] + b1_ref[...]
    h = jnp.maximum(h, 0.0)
    g2 = jnp.dot(h, w_ref[...], preferred_element_type=jnp.float32)
    out_ref[...] = g2 * rs_ref[...]


def _tc_mid(accA, accB, rd, rs, b1, W2):
    grid = NP // BR
    return pl.pallas_call(
        _mid_body,
        grid=(grid,),
        in_specs=[
            pl.BlockSpec((BR, D), lambda i: (i, 0)),
            pl.BlockSpec((BR, D), lambda i: (i, 0)),
            pl.BlockSpec((BR, 1), lambda i: (i, 0)),
            pl.BlockSpec((BR, 1), lambda i: (i, 0)),
            pl.BlockSpec((D,), lambda i: (0,)),
            pl.BlockSpec((D, D), lambda i: (0, 0)),
        ],
        out_specs=pl.BlockSpec((BR, D), lambda i: (i, 0)),
        out_shape=jax.ShapeDtypeStruct((NP, D), jnp.float32),
    )(accA, accB, rd, rs, b1, W2)


def _post_body(a_ref, b_ref, rd_ref, b2_ref, out_ref):
    out_ref[...] = (a_ref[...] + b_ref[...]) * rd_ref[...] + b2_ref[...]


def _tc_post(accA, accB, rd, b2):
    grid = NP // BR
    return pl.pallas_call(
        _post_body,
        grid=(grid,),
        in_specs=[
            pl.BlockSpec((BR, D), lambda i: (i, 0)),
            pl.BlockSpec((BR, D), lambda i: (i, 0)),
            pl.BlockSpec((BR, 1), lambda i: (i, 0)),
            pl.BlockSpec((D,), lambda i: (0,)),
        ],
        out_specs=pl.BlockSpec((BR, D), lambda i: (i, 0)),
        out_shape=jax.ShapeDtypeStruct((NP, D), jnp.float32),
    )(accA, accB, rd, b2)


def _decode_body(emb_hbm, idx_hbm, out_hbm,
                 sidx, didx, srows, drows, olocal, sem):
    c = lax.axis_index("c")
    s = lax.axis_index("s")
    wid = c * NS + s
    P = PAIRS_PER_TILE

    pltpu.sync_copy(idx_hbm.at[0, wid], sidx)
    pltpu.sync_copy(idx_hbm.at[1, wid], didx)
    pltpu.async_copy(emb_hbm.at[sidx], srows, sem).wait()
    pltpu.async_copy(emb_hbm.at[didx], drows, sem).wait()

    for pg in range(P // L):
        row_idx = pg * L + jnp.arange(L, dtype=jnp.int32)

        def dim_body(d, acc):
            col = jnp.full((L,), d, dtype=jnp.int32)
            sv = plsc.load_gather(srows, [row_idx, col])
            dv = plsc.load_gather(drows, [row_idx, col])
            return acc + sv * dv

        out16 = lax.fori_loop(0, D, dim_body,
                              jnp.zeros((L,), jnp.float32))
        olocal[pl.ds(pg * L, L)] = out16
    pltpu.sync_copy(olocal, out_hbm.at[pl.ds(wid * P, P)])


def _sc_decode(emb, idxp):
    P = PAIRS_PER_TILE
    return pl.kernel(
        _decode_body,
        out_type=jax.ShapeDtypeStruct((B,), jnp.float32),
        mesh=_sc_mesh(),
        compiler_params=_SC_PARAMS,
        scratch_types=[
            pltpu.VMEM((P,), jnp.int32),
            pltpu.VMEM((P,), jnp.int32),
            pltpu.VMEM((P, D), jnp.float32),
            pltpu.VMEM((P, D), jnp.float32),
            pltpu.VMEM((P,), jnp.float32),
            pltpu.SemaphoreType.DMA,
        ],
    )(emb, idxp)


@jax.jit
def kernel(x, edge_index, edge_weight, index, W1, b1, W2, b2):
    src = edge_index[0].astype(jnp.int32)
    dst = edge_index[1].astype(jnp.int32)
    w = edge_weight.astype(jnp.float32)

    pad = EPAD - E
    srcp = jnp.pad(src, (0, pad)).reshape(NW, NCH, C)
    dstp = jnp.pad(dst, (0, pad)).reshape(NW, NCH, C)
    wp = jnp.pad(w, (0, pad)).reshape(NW, NCH, C)
    xp = jnp.pad(x, ((0, NP - N), (0, 0)))
    idxp = index.astype(jnp.int32).reshape(2, NW, PAIRS_PER_TILE)

    xw1 = _tc_mm(xp, W1)
    degp = _sc_degrees(srcp, dstp, wp).reshape(NC, 2, NP)
    g1, r = _tc_rsqrt_scale(degp, xw1)
    rs = r[0].reshape(NP, 1)
    rd = r[1].reshape(NP, 1)

    acc1 = _sc_edge_pass(g1, srcp, dstp, wp)
    g2 = _tc_mid(acc1[0], acc1[1], rd, rs, b1, W2)
    acc2 = _sc_edge_pass(g2, srcp, dstp, wp)
    emb = _tc_post(acc2[0], acc2[1], rd, b2)
    out = _sc_decode(emb, idxp)
    return out
